# baseline (speedup 1.0000x reference)
import functools

import jax
import jax.numpy as jnp
from jax import lax
from jax.experimental import pallas as pl
from jax.experimental.pallas import tpu as pltpu
from jax.experimental.pallas import tpu_sc as plsc

BATCH = 16384
D = 64

_info = plsc.get_sparse_core_info()
NC, NS, L = _info.num_cores, _info.num_subcores, _info.num_lanes
NW = NC * NS
BPW = BATCH // NW
CHUNK = 128
NCHUNK = BPW // CHUNK


def _mf_body(uid_hbm, iid_hbm, ut_hbm, it_hbm, out_hbm,
             uidx_v, iidx_v, urows_v, irows_v, out_v, sem):
    wid = lax.axis_index("s") * NC + lax.axis_index("c")

    pltpu.sync_copy(uid_hbm.at[wid], uidx_v)
    pltpu.sync_copy(iid_hbm.at[wid], iidx_v)

    copies = []
    for c in range(NCHUNK):
        copies.append(pltpu.async_copy(
            ut_hbm.at[uidx_v.at[c]], urows_v.at[pl.ds(c * CHUNK, CHUNK)], sem))
        copies.append(pltpu.async_copy(
            it_hbm.at[iidx_v.at[c]], irows_v.at[pl.ds(c * CHUNK, CHUNK)], sem))
    for cp in copies:
        cp.wait()

    lanes = lax.broadcasted_iota(jnp.int32, (L,), 0)

    def group(g, carry):
        rows = g * L + lanes
        acc = jnp.zeros((L,), jnp.float32)
        for d in range(D):
            col = jnp.full((L,), d, jnp.int32)
            u = plsc.load_gather(urows_v, [rows, col])
            v = plsc.load_gather(irows_v, [rows, col])
            acc = acc + u * v
        out_v[pl.ds(pl.multiple_of(g * L, L), L)] = acc
        return carry

    lax.fori_loop(0, BPW // L, group, 0)

    pltpu.sync_copy(out_v, out_hbm.at[pl.ds(wid * BPW, BPW)])


@functools.partial(
    pl.kernel,
    out_type=jax.ShapeDtypeStruct((BATCH,), jnp.float32),
    mesh=plsc.VectorSubcoreMesh(core_axis_name="c", subcore_axis_name="s"),
    compiler_params=pltpu.CompilerParams(
        needs_layout_passes=False, use_tc_tiling_on_sc=False),
    scratch_types=[
        pltpu.VMEM((NCHUNK, CHUNK), jnp.int32),
        pltpu.VMEM((NCHUNK, CHUNK), jnp.int32),
        pltpu.VMEM((BPW, D), jnp.float32),
        pltpu.VMEM((BPW, D), jnp.float32),
        pltpu.VMEM((BPW,), jnp.float32),
        pltpu.SemaphoreType.DMA,
    ],
)
def _mf_kernel(uid_hbm, iid_hbm, ut_hbm, it_hbm, out_hbm,
               uidx_v, iidx_v, urows_v, irows_v, out_v, sem):
    _mf_body(uid_hbm, iid_hbm, ut_hbm, it_hbm, out_hbm,
             uidx_v, iidx_v, urows_v, irows_v, out_v, sem)


def kernel(user_ids, item_ids, user_table, item_table):
    uids = user_ids.astype(jnp.int32).reshape(NW, NCHUNK, CHUNK)
    iids = item_ids.astype(jnp.int32).reshape(NW, NCHUNK, CHUNK)
    return _mf_kernel(uids, iids, user_table, item_table)
